# baseline (speedup 1.0000x reference)
import functools

import jax
import jax.numpy as jnp
from jax import lax
from jax.experimental import pallas as pl
from jax.experimental.pallas import tpu as pltpu
from jax.experimental.pallas import tpu_sc as plsc

NC = 2
NS = 16
NW = NC * NS

N_ELEMS = 4 * 3 * 512 * 512
N_TRIPLES = N_ELEMS // 3
TRIPLES_PER_TILE = N_TRIPLES // NW
C = 2048
E = 3 * C
N_CHUNKS = TRIPLES_PER_TILE // C
ELEMS_PER_TILE = N_ELEMS // NW


def _sc_body(x_hbm, w_hbm, k_hbm, out_hbm, xv, idxf, wv, kv, outv, sem_w, sem_k):
    wid = lax.axis_index("s") * NC + lax.axis_index("c")
    base0 = wid * ELEMS_PER_TILE
    iota = lax.iota(jnp.int32, 16)

    def chunk_body(ci, _):
        e0 = base0 + ci * E
        pltpu.sync_copy(x_hbm.at[pl.ds(e0, E)], xv)

        def idx_body(it, _):
            p = it * 48 + iota * 3
            r = plsc.load_gather(xv, [p])
            g = plsc.load_gather(xv, [p + 1])
            b = plsc.load_gather(xv, [p + 2])
            ind = r * 65536.0 + g * 256.0 + b
            m = ind.astype(jnp.int32) * 3
            for j in range(3):
                plsc.store_scatter(idxf, [p + j], m + j)
            return 0

        lax.fori_loop(0, C // 16, idx_body, 0)

        cw = pltpu.async_copy(w_hbm.at[idxf], wv, sem_w)
        ck = pltpu.async_copy(k_hbm.at[idxf], kv, sem_k)
        cw.wait()
        ck.wait()

        def ew_body(u, _):
            sl = pl.ds(u * 16, 16)
            z = xv[sl] * wv[sl] + kv[sl]
            t = jnp.exp(-2.0 * jnp.abs(z))
            outv[sl] = jnp.sign(z) * ((1.0 - t) / (1.0 + t))
            return 0

        lax.fori_loop(0, E // 16, ew_body, 0)
        pltpu.sync_copy(outv, out_hbm.at[pl.ds(e0, E)])
        return 0

    lax.fori_loop(0, N_CHUNKS, chunk_body, 0)


@jax.jit
def _sc_call(xf, wf, kf):
    mesh = plsc.VectorSubcoreMesh(
        core_axis_name="c", subcore_axis_name="s",
        num_cores=NC, num_subcores=NS)
    f = pl.kernel(
        _sc_body,
        out_type=jax.ShapeDtypeStruct((N_ELEMS,), jnp.float32),
        mesh=mesh,
        scratch_types=[
            pltpu.VMEM((E,), jnp.float32),
            pltpu.VMEM((E,), jnp.int32),
            pltpu.VMEM((E,), jnp.float32),
            pltpu.VMEM((E,), jnp.float32),
            pltpu.VMEM((E,), jnp.float32),
            pltpu.SemaphoreType.DMA,
            pltpu.SemaphoreType.DMA,
        ],
        compiler_params=pltpu.CompilerParams(needs_layout_passes=False),
    )
    return f(xf, wf, kf)


def kernel(x, w, k):
    b, c, h, wd = x.shape
    out = _sc_call(x.reshape(-1), w.reshape(-1), k.reshape(-1))
    return out.reshape(-1, 3, h, wd)


# speedup vs baseline: 8.0309x; 8.0309x over previous
import jax
import jax.numpy as jnp
from jax import lax
from jax.experimental import pallas as pl
from jax.experimental.pallas import tpu as pltpu
from jax.experimental.pallas import tpu_sc as plsc

NC = 2
NS = 16
NW = NC * NS

TABLE_ROWS = 256 * 256 * 256

N_ELEMS = 4 * 3 * 512 * 512
N_TRIPLES = N_ELEMS // 3
TRIPLES_PER_TILE = N_TRIPLES // NW
C = 2048
E = 3 * C
N_CHUNKS = TRIPLES_PER_TILE // C
ELEMS_PER_TILE = N_ELEMS // NW


def _sc_body(x_hbm, w_hbm, k_hbm, out_hbm, xv, idxf, wv, kv, outv, sem_w, sem_k):
    wid = lax.axis_index("s") * NC + lax.axis_index("c")
    base0 = wid * ELEMS_PER_TILE
    iota = lax.iota(jnp.int32, 16)

    def chunk_body(ci, _):
        e0 = base0 + ci * E
        pltpu.sync_copy(x_hbm.at[pl.ds(e0, E)], xv)

        def idx_body(it, _):
            p = it * 48 + iota * 3
            r = plsc.load_gather(xv, [p])
            g = plsc.load_gather(xv, [p + 1])
            b = plsc.load_gather(xv, [p + 2])
            ind = (r * 65536.0 + g * 256.0 + b).astype(jnp.int32)
            for j in range(3):
                plsc.store_scatter(idxf, [p + j], ind + j * TABLE_ROWS)
            return 0

        lax.fori_loop(0, C // 16, idx_body, 0)

        cw = pltpu.async_copy(w_hbm.at[idxf], wv, sem_w)
        ck = pltpu.async_copy(k_hbm.at[idxf], kv, sem_k)
        cw.wait()
        ck.wait()

        def ew_body(u, _):
            sl = pl.ds(u * 16, 16)
            z = xv[sl] * wv[sl] + kv[sl]
            t = jnp.exp(-2.0 * jnp.abs(z))
            outv[sl] = jnp.sign(z) * ((1.0 - t) / (1.0 + t))
            return 0

        lax.fori_loop(0, E // 16, ew_body, 0)
        pltpu.sync_copy(outv, out_hbm.at[pl.ds(e0, E)])
        return 0

    lax.fori_loop(0, N_CHUNKS, chunk_body, 0)


@jax.jit
def _sc_call(xf, wp, kp):
    mesh = plsc.VectorSubcoreMesh(
        core_axis_name="c", subcore_axis_name="s",
        num_cores=NC, num_subcores=NS)
    f = pl.kernel(
        _sc_body,
        out_type=jax.ShapeDtypeStruct((N_ELEMS,), jnp.float32),
        mesh=mesh,
        scratch_types=[
            pltpu.VMEM((E,), jnp.float32),
            pltpu.VMEM((E,), jnp.int32),
            pltpu.VMEM((E,), jnp.float32),
            pltpu.VMEM((E,), jnp.float32),
            pltpu.VMEM((E,), jnp.float32),
            pltpu.SemaphoreType.DMA,
            pltpu.SemaphoreType.DMA,
        ],
        compiler_params=pltpu.CompilerParams(needs_layout_passes=False),
    )
    return f(xf, wp, kp)


def kernel(x, w, k):
    b, c, h, wd = x.shape
    out = _sc_call(x.reshape(-1), w.T.reshape(-1), k.T.reshape(-1))
    return out.reshape(-1, 3, h, wd)


# speedup vs baseline: 8.0334x; 1.0003x over previous
import jax
import jax.numpy as jnp
from jax import lax
from jax.experimental import pallas as pl
from jax.experimental.pallas import tpu as pltpu
from jax.experimental.pallas import tpu_sc as plsc

NC = 2
NS = 16
NW = NC * NS

TABLE_ROWS = 256 * 256 * 256

N_ELEMS = 4 * 3 * 512 * 512
N_TRIPLES = N_ELEMS // 3
TRIPLES_PER_TILE = N_TRIPLES // NW
C = 2048
E = 3 * C
N_CHUNKS = TRIPLES_PER_TILE // C
ELEMS_PER_TILE = N_ELEMS // NW


def _sc_body(x_hbm, w_hbm, k_hbm, out_hbm, xv, idxf, wv, kv, outv, sem_w, sem_k):
    wid = lax.axis_index("s") * NC + lax.axis_index("c")
    base0 = wid * ELEMS_PER_TILE
    iota = lax.iota(jnp.int32, 16)

    def chunk_body(ci, _):
        e0 = base0 + ci * E
        pltpu.sync_copy(x_hbm.at[pl.ds(e0, E)], xv)

        def idx_body(it, _):
            p = it * 48 + iota * 3
            r = plsc.load_gather(xv, [p])
            g = plsc.load_gather(xv, [p + 1])
            b = plsc.load_gather(xv, [p + 2])
            ind = (r * 65536.0 + g * 256.0 + b).astype(jnp.int32)
            for j in range(3):
                plsc.store_scatter(idxf, [p + j], ind + j * TABLE_ROWS)
            return 0

        lax.fori_loop(0, C // 16, idx_body, 0)

        cw = pltpu.async_copy(w_hbm.at[idxf], wv, sem_w)
        ck = pltpu.async_copy(k_hbm.at[idxf], kv, sem_k)
        cw.wait()
        ck.wait()

        def ew_body(u, _):
            sl = pl.ds(u * 16, 16)
            z = xv[sl] * wv[sl] + kv[sl]
            t = jnp.exp(-2.0 * jnp.abs(z))
            outv[sl] = jnp.sign(z) * ((1.0 - t) / (1.0 + t))
            return 0

        lax.fori_loop(0, E // 16, ew_body, 0)
        pltpu.sync_copy(outv, out_hbm.at[pl.ds(e0, E)])
        return 0

    lax.fori_loop(0, N_CHUNKS, chunk_body, 0)


@jax.jit
def _sc_call(xf, wp, kp):
    mesh = plsc.VectorSubcoreMesh(
        core_axis_name="c", subcore_axis_name="s",
        num_cores=NC, num_subcores=NS)
    f = pl.kernel(
        _sc_body,
        out_type=jax.ShapeDtypeStruct((N_ELEMS,), jnp.float32),
        mesh=mesh,
        scratch_types=[
            pltpu.VMEM((E,), jnp.float32),
            pltpu.VMEM((E,), jnp.int32),
            pltpu.VMEM((E,), jnp.float32),
            pltpu.VMEM((E,), jnp.float32),
            pltpu.VMEM((E,), jnp.float32),
            pltpu.SemaphoreType.DMA,
            pltpu.SemaphoreType.DMA,
        ],
        compiler_params=pltpu.CompilerParams(needs_layout_passes=False),
    )
    return f(xf, wp, kp)


def kernel(x, w, k):
    b, c, h, wd = x.shape
    wp = lax.reshape(w, (TABLE_ROWS * 3,), dimensions=(1, 0))
    kp = lax.reshape(k, (TABLE_ROWS * 3,), dimensions=(1, 0))
    out = _sc_call(x.reshape(-1), wp, kp)
    return out.reshape(-1, 3, h, wd)
